# Initial kernel scaffold; baseline (speedup 1.0000x reference)
import jax
import jax.numpy as jnp
from jax.experimental import pallas as pl


def kernel(uids, iids, labels, pos_rows, pos_cols, pos_vals, neg_rows, neg_cols, neg_vals, E_u_0, E_i_0, W0, b0, W1, b1, W2, b2):
    raise NotImplementedError("write your pallas kernel here")


# speedup vs baseline: 2.6178x; 2.6178x over previous
import functools

import jax
import jax.numpy as jnp
from jax import lax
from jax.experimental import pallas as pl
from jax.experimental.pallas import tpu as pltpu
from jax.experimental.pallas import tpu_sc as plsc

N_U = 5000
N_I = 5000
NP = 5120
D = 128
B = 16384
NNZ = 320000
CH = 128
E_PAD = 327680
NCH = E_PAD // (32 * CH)
RPT = NP // 16

_mesh = plsc.VectorSubcoreMesh(core_axis_name="c", subcore_axis_name="s")


def _spmm_body(gidx_hbm, sidx_hbm, vals_hbm, table_hbm, zeros_hbm, out_hbm,
               gidx_v, sidx_v, vals_v, G, acc, sem):
    cid = lax.axis_index("c")
    sid = lax.axis_index("s")
    wid = cid * 16 + sid
    base = wid * NCH

    pltpu.sync_copy(zeros_hbm, acc.at[pl.ds(sid * RPT, RPT)])
    pltpu.sync_copy(gidx_hbm.at[pl.ds(base, NCH)], gidx_v)
    pltpu.sync_copy(sidx_hbm.at[pl.ds(base, NCH)], sidx_v)
    pltpu.sync_copy(vals_hbm.at[pl.ds(base, NCH)], vals_v)
    plsc.subcore_barrier()

    def chunk(c, carry):
        pltpu.async_copy(table_hbm.at[gidx_v.at[c]], G, sem).wait()
        c16 = jnp.full((16,), c, jnp.int32)

        def scale(j, carry2):
            v = plsc.load_gather(vals_v, [c16, jnp.full((16,), j, jnp.int32)])
            for d in range(8):
                G[j, pl.ds(d * 16, 16)] = G[j, pl.ds(d * 16, 16)] * v
            return carry2

        lax.fori_loop(0, CH, scale, 0)
        pltpu.sync_copy(G, acc.at[sidx_v.at[c]], add=True)
        return carry

    lax.fori_loop(0, NCH, chunk, 0)
    plsc.subcore_barrier()
    r0 = sid * RPT
    pltpu.sync_copy(acc.at[pl.ds(r0, RPT)], out_hbm.at[cid, pl.ds(r0, RPT)])


_spmm = functools.partial(
    pl.kernel,
    out_type=jax.ShapeDtypeStruct((2, NP, D), jnp.float32),
    mesh=_mesh,
    scratch_types=[
        pltpu.VMEM((NCH, CH), jnp.int32),
        pltpu.VMEM((NCH, CH), jnp.int32),
        pltpu.VMEM((NCH, CH), jnp.float32),
        pltpu.VMEM((CH, D), jnp.float32),
        pltpu.VMEM_SHARED((NP, D), jnp.float32),
        pltpu.SemaphoreType.DMA,
    ],
    compiler_params=pltpu.CompilerParams(needs_layout_passes=False),
)(_spmm_body)


def _gather_body(eu_hbm, ei_hbm, uidx_hbm, iidx_hbm, ou_hbm, oi_hbm,
                 uix, iix, Gb, sem):
    cid = lax.axis_index("c")
    sid = lax.axis_index("s")
    wid = cid * 16 + sid
    pltpu.sync_copy(uidx_hbm.at[pl.ds(wid * 4, 4)], uix)
    pltpu.sync_copy(iidx_hbm.at[pl.ds(wid * 4, 4)], iix)

    def go(c, carry):
        pltpu.async_copy(eu_hbm.at[uix.at[c]], Gb, sem).wait()
        pltpu.sync_copy(Gb, ou_hbm.at[pl.ds(wid * 512 + c * CH, CH)])
        pltpu.async_copy(ei_hbm.at[iix.at[c]], Gb, sem).wait()
        pltpu.sync_copy(Gb, oi_hbm.at[pl.ds(wid * 512 + c * CH, CH)])
        return carry

    lax.fori_loop(0, 4, go, 0)


_gather = functools.partial(
    pl.kernel,
    out_type=(jax.ShapeDtypeStruct((B, D), jnp.float32),
              jax.ShapeDtypeStruct((B, D), jnp.float32)),
    mesh=_mesh,
    scratch_types=[
        pltpu.VMEM((4, CH), jnp.int32),
        pltpu.VMEM((4, CH), jnp.int32),
        pltpu.VMEM((CH, D), jnp.float32),
        pltpu.SemaphoreType.DMA,
    ],
    compiler_params=pltpu.CompilerParams(needs_layout_passes=False),
)(_gather_body)


_BLK = 512


def _dense_body(pu_ref, pi_ref, eu_ref, ei_ref, w_ref, b_ref, ou_ref, oi_ref,
                *, normalize_u):
    w = w_ref[...]
    bb = b_ref[...]

    def one(p_ref, e_ref, o_ref, norm):
        x = p_ref[0] + p_ref[1] + e_ref[...]
        y = lax.dot_general(x, w, (((1,), (1,)), ((), ())),
                            preferred_element_type=jnp.float32) + bb
        y = jnp.where(y >= 0, y, 0.2 * y)
        if norm:
            n = jnp.sqrt(jnp.sum(y * y, axis=1, keepdims=True))
            y = y / jnp.clip(n, 1e-12, None)
        o_ref[...] = y

    one(pu_ref, eu_ref, ou_ref, normalize_u)
    one(pi_ref, ei_ref, oi_ref, False)


def _dense(pu, pi, eu, ei, w, b, normalize_u):
    body = functools.partial(_dense_body, normalize_u=normalize_u)
    return pl.pallas_call(
        body,
        grid=(NP // _BLK,),
        in_specs=[
            pl.BlockSpec((2, _BLK, D), lambda i: (0, i, 0)),
            pl.BlockSpec((2, _BLK, D), lambda i: (0, i, 0)),
            pl.BlockSpec((_BLK, D), lambda i: (i, 0)),
            pl.BlockSpec((_BLK, D), lambda i: (i, 0)),
            pl.BlockSpec((D, D), lambda i: (0, 0)),
            pl.BlockSpec((1, D), lambda i: (0, 0)),
        ],
        out_specs=[
            pl.BlockSpec((_BLK, D), lambda i: (i, 0)),
            pl.BlockSpec((_BLK, D), lambda i: (i, 0)),
        ],
        out_shape=[jax.ShapeDtypeStruct((NP, D), jnp.float32),
                   jax.ShapeDtypeStruct((NP, D), jnp.float32)],
    )(pu, pi, eu, ei, w, b)


def _loss_body(u_ref, i_ref, lab_ref, logits_ref, loss_ref, accs):
    k = pl.program_id(0)
    u = u_ref[...]
    v = i_ref[...]
    prod = jnp.sum(u * v, axis=1)
    logits_ref[...] = prod.reshape(8, 128)
    lab = lab_ref[...].reshape(1024)
    bce = (jnp.maximum(prod, 0.0) - prod * lab
           + jnp.log1p(jnp.exp(-jnp.abs(prod))))
    reg = jnp.sum(u * u) + jnp.sum(v * v)

    @pl.when(k == 0)
    def _():
        accs[0] = 0.0
        accs[1] = 0.0

    accs[0] = accs[0] + jnp.sum(bce)
    accs[1] = accs[1] + reg

    @pl.when(k == pl.num_programs(0) - 1)
    def _():
        loss_ref[...] = jnp.full((1, 1), accs[0] / B + 1e-6 * accs[1],
                                 jnp.float32)


def _loss(u_emb, i_emb, labels2d):
    return pl.pallas_call(
        _loss_body,
        grid=(16,),
        in_specs=[
            pl.BlockSpec((1024, D), lambda i: (i, 0)),
            pl.BlockSpec((1024, D), lambda i: (i, 0)),
            pl.BlockSpec((8, 128), lambda i: (i, 0)),
        ],
        out_specs=[
            pl.BlockSpec((8, 128), lambda i: (i, 0)),
            pl.BlockSpec((1, 1), lambda i: (0, 0)),
        ],
        out_shape=[jax.ShapeDtypeStruct((128, 128), jnp.float32),
                   jax.ShapeDtypeStruct((1, 1), jnp.float32)],
        scratch_shapes=[pltpu.SMEM((2,), jnp.float32)],
    )(u_emb, i_emb, labels2d)


def kernel(uids, iids, labels, pos_rows, pos_cols, pos_vals,
           neg_rows, neg_cols, neg_vals, E_u_0, E_i_0,
           W0, b0, W1, b1, W2, b2):
    pad = E_PAD - NNZ
    rows2d = jnp.pad(jnp.concatenate([pos_rows, neg_rows]),
                     (0, pad)).reshape(E_PAD // CH, CH)
    cols2d = jnp.pad(jnp.concatenate([pos_cols, neg_cols]),
                     (0, pad)).reshape(E_PAD // CH, CH)
    vals2d = jnp.pad(jnp.concatenate([pos_vals, -neg_vals]),
                     (0, pad)).reshape(E_PAD // CH, CH)
    z = jnp.zeros((RPT, D), jnp.float32)
    eu = jnp.pad(E_u_0, ((0, NP - N_U), (0, 0)))
    ei = jnp.pad(E_i_0, ((0, NP - N_I), (0, 0)))

    for li, (W, b) in enumerate(((W0, b0), (W1, b1), (W2, b2))):
        pu = _spmm(cols2d, rows2d, vals2d, ei, z)
        pi = _spmm(rows2d, cols2d, vals2d, eu, z)
        eu, ei = _dense(pu, pi, eu, ei, W, b.reshape(1, D),
                        normalize_u=(li == 2))

    u_emb, i_emb = _gather(eu, ei, uids.reshape(B // CH, CH),
                           iids.reshape(B // CH, CH))
    logits2d, loss11 = _loss(u_emb, i_emb, labels.reshape(128, 128))
    return loss11.reshape(()), logits2d.reshape(B)


# speedup vs baseline: 3.4770x; 1.3282x over previous
import functools

import jax
import jax.numpy as jnp
from jax import lax
from jax.experimental import pallas as pl
from jax.experimental.pallas import tpu as pltpu
from jax.experimental.pallas import tpu_sc as plsc

N_U = 5000
N_I = 5000
NP = 5120
D = 128
B = 16384
NNZ = 320000
CH = 128
SCH = 80
E_PAD = 327680
NCH = E_PAD // (32 * SCH)
RPT = NP // 16

_mesh = plsc.VectorSubcoreMesh(core_axis_name="c", subcore_axis_name="s")


_NBUF = 4


def _spmm_body(gidx_hbm, sidx_hbm, vals_hbm, table_hbm, zeros_hbm, out_hbm,
               gidx_v, sidx_v, vals_v, G, acc,
               sg0, sg1, sg2, sg3, ss0, ss1, ss2, ss3):
    sg = (sg0, sg1, sg2, sg3)
    ss = (ss0, ss1, ss2, ss3)
    cid = lax.axis_index("c")
    sid = lax.axis_index("s")
    wid = cid * 16 + sid
    base = wid * NCH

    pltpu.sync_copy(zeros_hbm, acc.at[pl.ds(sid * RPT, RPT)])
    pltpu.sync_copy(gidx_hbm.at[pl.ds(base, NCH)], gidx_v)
    pltpu.sync_copy(sidx_hbm.at[pl.ds(base, NCH)], sidx_v)
    pltpu.sync_copy(vals_hbm.at[pl.ds(base, NCH)], vals_v)
    plsc.subcore_barrier()

    def start_gather(c, k):
        pltpu.async_copy(table_hbm.at[gidx_v.at[c]], G.at[k], sg[k])

    def wait_gather(c, k):
        pltpu.make_async_copy(table_hbm.at[gidx_v.at[c]], G.at[k],
                              sg[k]).wait()

    def start_scatter(c, k):
        pltpu.async_copy(G.at[k], acc.at[sidx_v.at[c]], ss[k], add=True)

    def wait_scatter(c, k):
        pltpu.make_async_copy(G.at[k], acc.at[sidx_v.at[c]], ss[k]).wait()

    def scale_chunk(c, k):
        c16 = jnp.full((16,), c, jnp.int32)

        def sc4(j4, carry):
            for u in range(4):
                j = j4 * 4 + u
                v = plsc.load_gather(
                    vals_v, [c16, jnp.full((16,), j, jnp.int32)])
                for d in range(8):
                    G[k, j, pl.ds(d * 16, 16)] = (
                        G[k, j, pl.ds(d * 16, 16)] * v)
            return carry

        lax.fori_loop(0, SCH // 4, sc4, 0)

    start_gather(0, 0)
    start_gather(1, 1)

    def outer(q, carry):
        for k in range(_NBUF):
            c = q * _NBUF + k
            k2 = (k + 2) % _NBUF

            @pl.when(c + 2 < NCH)
            def _():
                @pl.when(c >= 2)
                def _():
                    wait_scatter(c - 2, k2)
                start_gather(c + 2, k2)

            wait_gather(c, k)
            scale_chunk(c, k)
            start_scatter(c, k)
        return carry

    lax.fori_loop(0, NCH // _NBUF, outer, 0)
    for k in range(_NBUF):
        wait_scatter(NCH - _NBUF + k, k)
    plsc.subcore_barrier()
    r0 = sid * RPT
    pltpu.sync_copy(acc.at[pl.ds(r0, RPT)], out_hbm.at[cid, pl.ds(r0, RPT)])


_spmm = functools.partial(
    pl.kernel,
    out_type=jax.ShapeDtypeStruct((2, NP, D), jnp.float32),
    mesh=_mesh,
    scratch_types=[
        pltpu.VMEM((NCH, SCH), jnp.int32),
        pltpu.VMEM((NCH, SCH), jnp.int32),
        pltpu.VMEM((NCH, SCH), jnp.float32),
        pltpu.VMEM((_NBUF, SCH, D), jnp.float32),
        pltpu.VMEM_SHARED((NP, D), jnp.float32),
    ] + [pltpu.SemaphoreType.DMA] * 8,
    compiler_params=pltpu.CompilerParams(needs_layout_passes=False),
)(_spmm_body)


def _gather_body(eu_hbm, ei_hbm, uidx_hbm, iidx_hbm, ou_hbm, oi_hbm,
                 uix, iix, Gb, sem):
    cid = lax.axis_index("c")
    sid = lax.axis_index("s")
    wid = cid * 16 + sid
    pltpu.sync_copy(uidx_hbm.at[pl.ds(wid * 4, 4)], uix)
    pltpu.sync_copy(iidx_hbm.at[pl.ds(wid * 4, 4)], iix)

    def go(c, carry):
        pltpu.async_copy(eu_hbm.at[uix.at[c]], Gb, sem).wait()
        pltpu.sync_copy(Gb, ou_hbm.at[pl.ds(wid * 512 + c * CH, CH)])
        pltpu.async_copy(ei_hbm.at[iix.at[c]], Gb, sem).wait()
        pltpu.sync_copy(Gb, oi_hbm.at[pl.ds(wid * 512 + c * CH, CH)])
        return carry

    lax.fori_loop(0, 4, go, 0)


_gather = functools.partial(
    pl.kernel,
    out_type=(jax.ShapeDtypeStruct((B, D), jnp.float32),
              jax.ShapeDtypeStruct((B, D), jnp.float32)),
    mesh=_mesh,
    scratch_types=[
        pltpu.VMEM((4, CH), jnp.int32),
        pltpu.VMEM((4, CH), jnp.int32),
        pltpu.VMEM((CH, D), jnp.float32),
        pltpu.SemaphoreType.DMA,
    ],
    compiler_params=pltpu.CompilerParams(needs_layout_passes=False),
)(_gather_body)


_BLK = 512


def _dense_body(pu_ref, pi_ref, eu_ref, ei_ref, w_ref, b_ref, ou_ref, oi_ref,
                *, normalize_u):
    w = w_ref[...]
    bb = b_ref[...]

    def one(p_ref, e_ref, o_ref, norm):
        x = p_ref[0] + p_ref[1] + e_ref[...]
        y = lax.dot_general(x, w, (((1,), (1,)), ((), ())),
                            preferred_element_type=jnp.float32) + bb
        y = jnp.where(y >= 0, y, 0.2 * y)
        if norm:
            n = jnp.sqrt(jnp.sum(y * y, axis=1, keepdims=True))
            y = y / jnp.clip(n, 1e-12, None)
        o_ref[...] = y

    one(pu_ref, eu_ref, ou_ref, normalize_u)
    one(pi_ref, ei_ref, oi_ref, False)


def _dense(pu, pi, eu, ei, w, b, normalize_u):
    body = functools.partial(_dense_body, normalize_u=normalize_u)
    return pl.pallas_call(
        body,
        grid=(NP // _BLK,),
        in_specs=[
            pl.BlockSpec((2, _BLK, D), lambda i: (0, i, 0)),
            pl.BlockSpec((2, _BLK, D), lambda i: (0, i, 0)),
            pl.BlockSpec((_BLK, D), lambda i: (i, 0)),
            pl.BlockSpec((_BLK, D), lambda i: (i, 0)),
            pl.BlockSpec((D, D), lambda i: (0, 0)),
            pl.BlockSpec((1, D), lambda i: (0, 0)),
        ],
        out_specs=[
            pl.BlockSpec((_BLK, D), lambda i: (i, 0)),
            pl.BlockSpec((_BLK, D), lambda i: (i, 0)),
        ],
        out_shape=[jax.ShapeDtypeStruct((NP, D), jnp.float32),
                   jax.ShapeDtypeStruct((NP, D), jnp.float32)],
    )(pu, pi, eu, ei, w, b)


def _loss_body(u_ref, i_ref, lab_ref, logits_ref, loss_ref, accs):
    k = pl.program_id(0)
    u = u_ref[...]
    v = i_ref[...]
    prod = jnp.sum(u * v, axis=1)
    logits_ref[...] = prod.reshape(8, 128)
    lab = lab_ref[...].reshape(1024)
    bce = (jnp.maximum(prod, 0.0) - prod * lab
           + jnp.log1p(jnp.exp(-jnp.abs(prod))))
    reg = jnp.sum(u * u) + jnp.sum(v * v)

    @pl.when(k == 0)
    def _():
        accs[0] = 0.0
        accs[1] = 0.0

    accs[0] = accs[0] + jnp.sum(bce)
    accs[1] = accs[1] + reg

    @pl.when(k == pl.num_programs(0) - 1)
    def _():
        loss_ref[...] = jnp.full((1, 1), accs[0] / B + 1e-6 * accs[1],
                                 jnp.float32)


def _loss(u_emb, i_emb, labels2d):
    return pl.pallas_call(
        _loss_body,
        grid=(16,),
        in_specs=[
            pl.BlockSpec((1024, D), lambda i: (i, 0)),
            pl.BlockSpec((1024, D), lambda i: (i, 0)),
            pl.BlockSpec((8, 128), lambda i: (i, 0)),
        ],
        out_specs=[
            pl.BlockSpec((8, 128), lambda i: (i, 0)),
            pl.BlockSpec((1, 1), lambda i: (0, 0)),
        ],
        out_shape=[jax.ShapeDtypeStruct((128, 128), jnp.float32),
                   jax.ShapeDtypeStruct((1, 1), jnp.float32)],
        scratch_shapes=[pltpu.SMEM((2,), jnp.float32)],
    )(u_emb, i_emb, labels2d)


def kernel(uids, iids, labels, pos_rows, pos_cols, pos_vals,
           neg_rows, neg_cols, neg_vals, E_u_0, E_i_0,
           W0, b0, W1, b1, W2, b2):
    pad = E_PAD - NNZ
    rows2d = jnp.pad(jnp.concatenate([pos_rows, neg_rows]),
                     (0, pad)).reshape(E_PAD // SCH, SCH)
    cols2d = jnp.pad(jnp.concatenate([pos_cols, neg_cols]),
                     (0, pad)).reshape(E_PAD // SCH, SCH)
    vals2d = jnp.pad(jnp.concatenate([pos_vals, -neg_vals]),
                     (0, pad)).reshape(E_PAD // SCH, SCH)
    z = jnp.zeros((RPT, D), jnp.float32)
    eu = jnp.pad(E_u_0, ((0, NP - N_U), (0, 0)))
    ei = jnp.pad(E_i_0, ((0, NP - N_I), (0, 0)))

    for li, (W, b) in enumerate(((W0, b0), (W1, b1), (W2, b2))):
        pu = _spmm(cols2d, rows2d, vals2d, ei, z)
        pi = _spmm(rows2d, cols2d, vals2d, eu, z)
        eu, ei = _dense(pu, pi, eu, ei, W, b.reshape(1, D),
                        normalize_u=(li == 2))

    u_emb, i_emb = _gather(eu, ei, uids.reshape(B // CH, CH),
                           iids.reshape(B // CH, CH))
    logits2d, loss11 = _loss(u_emb, i_emb, labels.reshape(128, 128))
    return loss11.reshape(()), logits2d.reshape(B)
